# baseline (speedup 1.0000x reference)
import functools

import jax
import jax.numpy as jnp
from jax import lax
from jax.experimental import pallas as pl
from jax.experimental.pallas import tpu as pltpu
from jax.experimental.pallas import tpu_sc as plsc

_IDX_MID = (2, 0, 0, 1, 1, 3, 3, 5, 4, 5, 4, 7, 6, 7, 6, 9, 8, 11, 11, 9,
            10, 10, 8, 12, 12, 13, 13, 14, 14, 15, 15, 16, 17, 17, 18, 18,
            19, 19, 20, 20, 21, 21, 22, 22, 23, 23, 24, 24, 25, 25, 26, 26,
            27, 27, 28, 28, 29, 30, 30, 31, 32, 32, 31, 33, 33, 34, 35, 35,
            34, 36, 36, 37, 37, 38, 29, 38, 39, 39, 40, 40, 16, 41, 41, 42,
            42, 43, 43, 44, 44, 45, 45, 46, 47, 47, 48, 48, 49, 49, 50, 50,
            51, 51, 52, 52, 53, 53, 54, 54, 55, 55, 56, 56, 57, 58, 58, 59,
            59, 60, 60, 61, 61, 62, 62, 63, 63, 64, 64, 65, 65, 66, 66, 67,
            67)

_IDX_LOW = (0, 0, 1, 1, 2, 2, 3, 3, 4, 4, 5, 5, 6, 7, 8, 9, 10, 9, 8, 7,
            6, 11, 12, 13, 12, 11, 13, 14, 15, 14, 15, 16, 17, 18, 16, 17,
            18, 19, 10, 19, 20, 20, 21, 21, 22, 22, 23, 24, 25, 26, 27, 28,
            29, 30, 31, 32, 33, 23, 24, 25, 26, 27, 28, 29, 30, 31, 32, 33)

_NBUF = 4


@functools.lru_cache(maxsize=None)
def _make_unpool(batch, j_in, d, idx):
    j_out = len(idx)
    info = plsc.get_sparse_core_info()
    nc = info.num_cores
    nw = nc * info.num_subcores
    assert batch % nw == 0
    bb = batch // nw

    outs = [[j for j in range(j_out) if idx[j] == i] for i in range(j_in)]

    @functools.partial(
        pl.kernel,
        out_type=jax.ShapeDtypeStruct((batch, j_out, d), jnp.float32),
        mesh=plsc.VectorSubcoreMesh(core_axis_name="c", subcore_axis_name="s"),
        scratch_types=[pltpu.VMEM((bb, 1, d), jnp.float32) for _ in range(_NBUF)]
        + [pltpu.SemaphoreType.DMA for _ in range(2 * _NBUF)],
        compiler_params=pltpu.CompilerParams(use_tc_tiling_on_sc=True),
    )
    def unpool(x_hbm, out_hbm, *rest):
        bufs = rest[:_NBUF]
        gsems = rest[_NBUF:2 * _NBUF]
        ssems = rest[2 * _NBUF:]
        wid = lax.axis_index("s") * nc + lax.axis_index("c")
        b0 = wid * bb

        def gath(i, k):
            return pltpu.make_async_copy(
                x_hbm.at[pl.ds(b0, bb), pl.ds(i, 1), :], bufs[k], gsems[k])

        def scat(j, k):
            return pltpu.make_async_copy(
                bufs[k], out_hbm.at[pl.ds(b0, bb), pl.ds(j, 1), :], ssems[k])

        gath(0, 0).start()
        gath(1, 1).start()
        for i in range(j_in):
            k = i % _NBUF
            gath(i, k).wait()
            for j in outs[i]:
                scat(j, k).start()
            if i >= 2:
                k2 = (i - 2) % _NBUF
                for j in outs[i - 2]:
                    scat(j, k2).wait()
            if i + 2 < j_in:
                gath(i + 2, (i + 2) % _NBUF).start()
        for i in (j_in - 2, j_in - 1):
            for j in outs[i]:
                scat(j, i % _NBUF).wait()

    return unpool


def kernel(x):
    batch, j_in, d = x.shape
    idx = _IDX_MID if j_in == 68 else _IDX_LOW
    return _make_unpool(batch, j_in, d, idx)(x)


# speedup vs baseline: 1.0902x; 1.0902x over previous
import functools

import jax
import jax.numpy as jnp
from jax import lax
from jax.experimental import pallas as pl
from jax.experimental.pallas import tpu as pltpu

_IDX_MID = (2, 0, 0, 1, 1, 3, 3, 5, 4, 5, 4, 7, 6, 7, 6, 9, 8, 11, 11, 9,
            10, 10, 8, 12, 12, 13, 13, 14, 14, 15, 15, 16, 17, 17, 18, 18,
            19, 19, 20, 20, 21, 21, 22, 22, 23, 23, 24, 24, 25, 25, 26, 26,
            27, 27, 28, 28, 29, 30, 30, 31, 32, 32, 31, 33, 33, 34, 35, 35,
            34, 36, 36, 37, 37, 38, 29, 38, 39, 39, 40, 40, 16, 41, 41, 42,
            42, 43, 43, 44, 44, 45, 45, 46, 47, 47, 48, 48, 49, 49, 50, 50,
            51, 51, 52, 52, 53, 53, 54, 54, 55, 55, 56, 56, 57, 58, 58, 59,
            59, 60, 60, 61, 61, 62, 62, 63, 63, 64, 64, 65, 65, 66, 66, 67,
            67)

_IDX_LOW = (0, 0, 1, 1, 2, 2, 3, 3, 4, 4, 5, 5, 6, 7, 8, 9, 10, 9, 8, 7,
            6, 11, 12, 13, 12, 11, 13, 14, 15, 14, 15, 16, 17, 18, 16, 17,
            18, 19, 10, 19, 20, 20, 21, 21, 22, 22, 23, 24, 25, 26, 27, 28,
            29, 30, 31, 32, 33, 23, 24, 25, 26, 27, 28, 29, 30, 31, 32, 33)

_BB = 256


@functools.lru_cache(maxsize=None)
def _make_unpool(batch, j_in, d, idx):
    j_out = len(idx)
    assert batch % (2 * _BB) == 0
    nblk = batch // _BB

    def body(x_hbm, out_hbm, slab0, slab1, rsem0, rsem1, wsem0, wsem1):
        slabs = (slab0, slab1)
        rsems = (rsem0, rsem1)
        wsems = (wsem0, wsem1)

        def read(p, k):
            return pltpu.make_async_copy(
                x_hbm.at[pl.ds(p * _BB, _BB)], slabs[k], rsems[k])

        def write(p, j, k):
            return pltpu.make_async_copy(
                slabs[k].at[:, pl.ds(idx[j], 1), :],
                out_hbm.at[pl.ds(p * _BB, _BB), pl.ds(j, 1), :],
                wsems[k])

        read(0, 0).start()

        def loop(i, carry):
            for k in range(2):
                p = i * 2 + k
                read(p, k).wait()

                @pl.when(p + 1 < nblk)
                def _():
                    @pl.when(p >= 1)
                    def _():
                        for j in range(j_out):
                            write(p - 1, j, 1 - k).wait()

                    read(p + 1, 1 - k).start()

                for j in range(j_out):
                    write(p, j, k).start()
            return carry

        lax.fori_loop(0, nblk // 2, loop, 0)
        for p in (nblk - 2, nblk - 1):
            for j in range(j_out):
                write(p, j, p % 2).wait()

    return pl.pallas_call(
        body,
        out_shape=jax.ShapeDtypeStruct((batch, j_out, d), jnp.float32),
        in_specs=[pl.BlockSpec(memory_space=pl.ANY)],
        out_specs=pl.BlockSpec(memory_space=pl.ANY),
        scratch_shapes=[
            pltpu.VMEM((_BB, j_in, d), jnp.float32),
            pltpu.VMEM((_BB, j_in, d), jnp.float32),
            pltpu.SemaphoreType.DMA,
            pltpu.SemaphoreType.DMA,
            pltpu.SemaphoreType.DMA,
            pltpu.SemaphoreType.DMA,
        ],
    )


def kernel(x):
    batch, j_in, d = x.shape
    idx = _IDX_MID if j_in == 68 else _IDX_LOW
    return _make_unpool(batch, j_in, d, idx)(x)
